# baseline (speedup 1.0000x reference)
import functools

import jax
import jax.numpy as jnp
from jax import lax
from jax.experimental import pallas as pl
from jax.experimental.pallas import tpu as pltpu
from jax.experimental.pallas import tpu_sc as plsc

_DIM = 1024
_NC = 2
_NS = 16
_NW = _NC * _NS
_CHUNK = 32


def _emb_body(total, bpw, nchunk,
              idx_hbm, table_hbm, out_hbm,
              idx_v, rows, gs, ss):
    wid = lax.axis_index("s") * _NC + lax.axis_index("c")
    base = wid * bpw

    pltpu.sync_copy(idx_hbm.at[pl.ds(base, bpw)], idx_v)

    def gather(chunk, b):
        src = table_hbm.at[idx_v.at[pl.ds(chunk * _CHUNK, _CHUNK)]]
        return pltpu.make_async_copy(src, rows[b], gs[b])

    def store(chunk, b):
        dst = out_hbm.at[pl.ds(base + chunk * _CHUNK, _CHUNK)]
        return pltpu.make_async_copy(rows[b], dst, ss[b])

    gather(0, 0).start()
    gather(1, 1).start()

    ngroup = (nchunk - 2) // 3

    def group(g, _):
        for j in range(3):
            c = 3 * g + j
            bn = (j + 2) % 3
            gather(c, j).wait()
            store(c, j).start()

            @pl.when(c >= 1)
            def _():
                store(c - 1, bn).wait()

            gather(c + 2, bn).start()
        return None

    lax.fori_loop(0, ngroup, group, None, unroll=False)

    for c in range(3 * ngroup, nchunk):
        b = c % 3
        gather(c, b).wait()
        store(c - 1, (b + 2) % 3).wait()
        store(c, b).start()
    store(nchunk - 1, (nchunk - 1) % 3).wait()


def kernel(pos_ids, table):
    batch, seq = pos_ids.shape
    dim = table.shape[1]
    total = batch * seq
    nchunk = 23
    bpw = nchunk * _CHUNK
    sc_total = bpw * _NW

    all_ids = pos_ids.reshape(total).astype(jnp.int32)
    flat_ids = all_ids[:sc_total]
    tc_out = jnp.take(table, all_ids[sc_total:], axis=0)

    mesh = plsc.VectorSubcoreMesh(core_axis_name="c", subcore_axis_name="s")
    body = functools.partial(_emb_body, total, bpw, nchunk)
    out = pl.kernel(
        body,
        out_type=jax.ShapeDtypeStruct((sc_total, dim), jnp.float32),
        mesh=mesh,
        scratch_types=[
            pltpu.VMEM((bpw,), jnp.int32),
            [pltpu.VMEM((_CHUNK, dim), jnp.float32) for _ in range(3)],
            [pltpu.SemaphoreType.DMA for _ in range(3)],
            [pltpu.SemaphoreType.DMA for _ in range(3)],
        ],
    )(flat_ids, table)
    return (out, tc_out)


# speedup vs baseline: 1.2697x; 1.2697x over previous
import functools

import jax
import jax.numpy as jnp
from jax import lax
from jax.experimental import pallas as pl
from jax.experimental.pallas import tpu as pltpu
from jax.experimental.pallas import tpu_sc as plsc

_NC = 2
_NS = 16
_NW = _NC * _NS
_CHUNK = 16
_NBUF = 7


def _emb_body(bpw, nchunk,
              idx_hbm, table_hbm, out_hbm,
              idx_v, rows, gs, ss):
    wid = lax.axis_index("s") * _NC + lax.axis_index("c")
    base = wid * bpw

    pltpu.sync_copy(idx_hbm.at[pl.ds(base, bpw)], idx_v)

    def gather(chunk, b):
        src = table_hbm.at[idx_v.at[pl.ds(chunk * _CHUNK, _CHUNK)]]
        return pltpu.make_async_copy(src, rows[b], gs[b])

    def store(chunk, b):
        dst = out_hbm.at[pl.ds(base + chunk * _CHUNK, _CHUNK)]
        return pltpu.make_async_copy(rows[b], dst, ss[b])

    for c in range(_NBUF - 1):
        gather(c, c).start()

    ngroup = (nchunk - 1) // _NBUF

    def group(g, _):
        for j in range(_NBUF):
            c = _NBUF * g + j
            bn = (j + _NBUF - 1) % _NBUF
            gather(c, j).wait()
            store(c, j).start()

            @pl.when((c >= 1) & (c + _NBUF - 1 < nchunk))
            def _():
                store(c - 1, bn).wait()

            @pl.when(c + _NBUF - 1 < nchunk)
            def _():
                gather(c + _NBUF - 1, bn).start()

        return None

    lax.fori_loop(0, ngroup, group, None, unroll=False)

    for c in range(_NBUF * ngroup, nchunk):
        gather(c, c % _NBUF).wait()
        store(c, c % _NBUF).start()
    for c in range(nchunk - _NBUF, nchunk):
        store(c, c % _NBUF).wait()


def kernel(pos_ids, table):
    batch, seq = pos_ids.shape
    dim = table.shape[1]
    total = batch * seq
    bpw = total // _NW
    nchunk = bpw // _CHUNK

    flat_ids = pos_ids.reshape(total).astype(jnp.int32)

    mesh = plsc.VectorSubcoreMesh(core_axis_name="c", subcore_axis_name="s")
    body = functools.partial(_emb_body, bpw, nchunk)
    out = pl.kernel(
        body,
        out_type=jax.ShapeDtypeStruct((total, dim), jnp.float32),
        mesh=mesh,
        scratch_types=[
            pltpu.VMEM((bpw,), jnp.int32),
            [pltpu.VMEM((_CHUNK, dim), jnp.float32) for _ in range(_NBUF)],
            [pltpu.SemaphoreType.DMA for _ in range(_NBUF)],
            [pltpu.SemaphoreType.DMA for _ in range(_NBUF)],
        ],
    )(flat_ids, table)
    return out.reshape(batch, seq, dim)
